# baseline (speedup 1.0000x reference)
import jax
import jax.numpy as jnp
from jax.experimental import pallas as pl
from jax.experimental.pallas import tpu as pltpu


def _half_block(n):
    for rb in (200, 100, 40, 8):
        if n % (2 * rb) == 0:
            return rb
    return n


def _dot_t(a, b):
    return jax.lax.dot_general(a, b, (((1,), (1,)), ((), ())),
                               preferred_element_type=jnp.float32)


def _dot(a, b):
    return jax.lax.dot_general(a, b, (((1,), (0,)), ((), ())),
                               preferred_element_type=jnp.float32)


def _cheb_kernel(th_ref, x_ref, la_ref, lb_ref, w1_ref, b1_ref, w2_ref,
                 b2_ref, out_ref, t0b_ref, t1b_ref, poly_ref):
    phase = pl.program_id(0)
    i = pl.program_id(1)
    rb = la_ref.shape[0]

    @pl.when((phase == 0) & (i == 0))
    def _fc1():
        h = _dot_t(x_ref[...].astype(jnp.bfloat16),
                   w1_ref[...].astype(jnp.bfloat16))
        h = jnp.maximum(h + b1_ref[...], 0.0)
        t0b_ref[...] = h.astype(jnp.bfloat16)

    @pl.when(phase == 0)
    def _prop1():
        for s, l_ref in ((0, la_ref), (1, lb_ref)):
            rows = pl.ds((2 * i + s) * rb, rb)
            t1 = _dot(l_ref[...].astype(jnp.bfloat16), t0b_ref[...])
            t1b_ref[rows, :] = t1.astype(jnp.bfloat16)
            poly_ref[rows, :] = (
                th_ref[0] * t0b_ref[rows, :].astype(jnp.float32)
                + (th_ref[1] + th_ref[2]) * t1)

    @pl.when(phase == 1)
    def _final():
        for s, l_ref in ((0, la_ref), (1, lb_ref)):
            rows = pl.ds((2 * i + s) * rb, rb)
            t3 = (2.0 * _dot(l_ref[...].astype(jnp.bfloat16), t1b_ref[...])
                  - t0b_ref[rows, :].astype(jnp.float32))
            p = poly_ref[rows, :] + th_ref[3] * t3
            y = _dot_t(p.astype(jnp.bfloat16),
                       w2_ref[...].astype(jnp.bfloat16))
            y = y + b2_ref[...]
            m = jnp.max(y, axis=1, keepdims=True)
            e = y - m
            lse = jnp.log(jnp.sum(jnp.exp(e), axis=1, keepdims=True))
            out_ref[pl.ds(s * rb, rb), :] = e - lse


def kernel(x, L, W1, b1, W2, b2, thetas):
    n, f = x.shape
    h = W1.shape[0]
    c = W2.shape[0]
    rb = _half_block(n)
    nblk = n // (2 * rb)
    b1r = b1.reshape(1, h)
    b2r = b2.reshape(1, c)

    def full(shape):
        return pl.BlockSpec(shape, lambda p, i: (0, 0))

    out = pl.pallas_call(
        _cheb_kernel,
        grid=(2, nblk),
        in_specs=[pl.BlockSpec(memory_space=pltpu.SMEM),
                  full((n, f)),
                  pl.BlockSpec((rb, n), lambda p, i: (2 * i, 0)),
                  pl.BlockSpec((rb, n), lambda p, i: (2 * i + 1, 0)),
                  full((h, f)), full((1, h)),
                  full((c, h)), full((1, c))],
        out_specs=pl.BlockSpec((2 * rb, c), lambda p, i: (p * i, 0)),
        out_shape=jax.ShapeDtypeStruct((n, c), jnp.float32),
        scratch_shapes=[pltpu.VMEM((n, h), jnp.bfloat16),
                        pltpu.VMEM((n, h), jnp.bfloat16),
                        pltpu.VMEM((n, h), jnp.float32)],
        compiler_params=pltpu.CompilerParams(
            dimension_semantics=("arbitrary", "arbitrary")),
    )(thetas, x, L, L, W1, b1r, W2, b2r)

    return out


# speedup vs baseline: 1.0148x; 1.0148x over previous
import jax
import jax.numpy as jnp
from jax.experimental import pallas as pl
from jax.experimental.pallas import tpu as pltpu


def _row_block(n):
    for rb in (400, 200, 80, 40, 8):
        if n % rb == 0:
            return rb
    return n


def _dot_t(a, b):
    return jax.lax.dot_general(a, b, (((1,), (1,)), ((), ())),
                               preferred_element_type=jnp.float32)


def _dot(a, b):
    return jax.lax.dot_general(a, b, (((1,), (0,)), ((), ())),
                               preferred_element_type=jnp.float32)


def _cheb_kernel(th_ref, x_ref, l_ref, w1_ref, b1_ref, w2_ref, b2_ref,
                 out_ref, t0b_ref, t1b_ref, poly_ref):
    phase = pl.program_id(0)
    i = pl.program_id(1)
    rb = l_ref.shape[0]
    rows = pl.ds(i * rb, rb)

    @pl.when((phase == 0) & (i == 0))
    def _fc1():
        h = _dot_t(x_ref[...].astype(jnp.bfloat16),
                   w1_ref[...].astype(jnp.bfloat16))
        h = jnp.maximum(h + b1_ref[...], 0.0)
        t0b_ref[...] = h.astype(jnp.bfloat16)

    @pl.when(phase == 0)
    def _prop1():
        t1 = _dot(l_ref[...].astype(jnp.bfloat16), t0b_ref[...])
        t1b_ref[rows, :] = t1.astype(jnp.bfloat16)
        poly_ref[rows, :] = (th_ref[0] * t0b_ref[rows, :].astype(jnp.float32)
                             + (th_ref[1] + th_ref[2]) * t1)

    @pl.when(phase == 1)
    def _final():
        t3 = (2.0 * _dot(l_ref[...].astype(jnp.bfloat16), t1b_ref[...])
              - t0b_ref[rows, :].astype(jnp.float32))
        p = poly_ref[rows, :] + th_ref[3] * t3
        y = _dot_t(p.astype(jnp.bfloat16), w2_ref[...].astype(jnp.bfloat16))
        y = y + b2_ref[...]
        m = jnp.max(y, axis=1, keepdims=True)
        e = y - m
        lse = jnp.log(jnp.sum(jnp.exp(e), axis=1, keepdims=True))
        out_ref[...] = e - lse


def kernel(x, L, W1, b1, W2, b2, thetas):
    n, f = x.shape
    h = W1.shape[0]
    c = W2.shape[0]
    rb = _row_block(n)
    nblk = n // rb
    b1r = b1.reshape(1, h)
    b2r = b2.reshape(1, c)

    def full(shape):
        return pl.BlockSpec(shape, lambda p, i: (0, 0))

    out = pl.pallas_call(
        _cheb_kernel,
        grid=(2, nblk),
        in_specs=[pl.BlockSpec(memory_space=pltpu.SMEM),
                  full((n, f)),
                  pl.BlockSpec((rb, n), lambda p, i: (i, 0)),
                  full((h, f)), full((1, h)),
                  full((c, h)), full((1, c))],
        out_specs=pl.BlockSpec((rb, c), lambda p, i: (p * i, 0)),
        out_shape=jax.ShapeDtypeStruct((n, c), jnp.float32),
        scratch_shapes=[pltpu.VMEM((n, h), jnp.bfloat16),
                        pltpu.VMEM((n, h), jnp.bfloat16),
                        pltpu.VMEM((n, h), jnp.float32)],
        compiler_params=pltpu.CompilerParams(
            dimension_semantics=("arbitrary", "arbitrary")),
    )(thetas, x, L, W1, b1r, W2, b2r)

    return out
